# baseline (speedup 1.0000x reference)
import functools

import jax
import jax.numpy as jnp
from jax import lax
from jax.experimental import pallas as pl
from jax.experimental.pallas import tpu as pltpu
from jax.experimental.pallas import tpu_sc as plsc

NC = 2
NS = 16
EW = 128
K0 = 128
K1 = 32
BTOT = NS * (K0 + K1)
P = 4
PH0 = K0 // P
PH1 = K1 // P
NB = 2


def _sc_mesh():
    return plsc.VectorSubcoreMesh(core_axis_name="c", subcore_axis_name="s")


def _make_hist_kernel(NPAD):
    rps = NPAD // NS
    KH = BTOT // (NC * NS)

    @functools.partial(
        pl.kernel,
        mesh=_sc_mesh(),
        out_type=(
            jax.ShapeDtypeStruct((NC, NPAD), jnp.float32),
            jax.ShapeDtypeStruct((NC, NPAD), jnp.float32),
        ),
        scratch_types=[
            pltpu.VMEM((KH, EW), jnp.int32),
            pltpu.VMEM((KH, EW), jnp.int32),
            pltpu.VMEM((EW,), jnp.float32),
            pltpu.VMEM_SHARED((NPAD,), jnp.float32),
            pltpu.VMEM_SHARED((NPAD,), jnp.float32),
        ],
    )
    def hist(src_hbm, dst_hbm, ones_hbm, z1_hbm, do_out, di_out,
             src_v, dst_v, ones_v, do_sh, di_sh):
        cid = lax.axis_index("c")
        sid = lax.axis_index("s")
        wid = sid * NC + cid
        pltpu.sync_copy(z1_hbm, do_sh.at[pl.ds(sid * rps, rps)])
        pltpu.sync_copy(z1_hbm, di_sh.at[pl.ds(sid * rps, rps)])
        pltpu.sync_copy(ones_hbm, ones_v)
        pltpu.sync_copy(src_hbm.at[pl.ds(wid * KH, KH)], src_v)
        pltpu.sync_copy(dst_hbm.at[pl.ds(wid * KH, KH)], dst_v)
        plsc.subcore_barrier()

        def body(j, carry):
            pltpu.sync_copy(ones_v, do_sh.at[src_v.at[j]], add=True)
            pltpu.sync_copy(ones_v, di_sh.at[dst_v.at[j]], add=True)
            return carry

        lax.fori_loop(0, KH, body, 0)
        plsc.subcore_barrier()
        pltpu.sync_copy(do_sh.at[pl.ds(sid * rps, rps)],
                        do_out.at[cid, pl.ds(sid * rps, rps)])
        pltpu.sync_copy(di_sh.at[pl.ds(sid * rps, rps)],
                        di_out.at[cid, pl.ds(sid * rps, rps)])

    return hist


def _make_edge_kernel(NPAD, Dh):
    rps = NPAD // NS

    @functools.partial(
        pl.kernel,
        mesh=_sc_mesh(),
        out_type=(
            jax.ShapeDtypeStruct((NC, NPAD, Dh), jnp.float32),
            jax.ShapeDtypeStruct((NC, NPAD), jnp.float32),
        ),
        scratch_types=[
            pltpu.VMEM((PH0, EW), jnp.int32),
            pltpu.VMEM((PH0, EW), jnp.int32),
            pltpu.VMEM((NB, EW, Dh), jnp.float32),
            pltpu.VMEM((PH0, EW), jnp.float32),
            pltpu.VMEM_SHARED((NPAD, Dh), jnp.float32),
            pltpu.VMEM_SHARED((NPAD,), jnp.float32),
            pltpu.VMEM_SHARED((NPAD,), jnp.float32),
            pltpu.SemaphoreType.DMA,
            pltpu.SemaphoreType.DMA,
            pltpu.SemaphoreType.DMA,
            pltpu.SemaphoreType.DMA,
        ],
    )
    def edge(src_hbm, dst_hbm, y0_hbm, y1_hbm, nd_hbm, z1_hbm,
             agg_out, c_out, src_v, dst_v, rows_v, ndv_v,
             agg_sh, c_sh, nd_sh, sem_y0, sem_y1, sem_n, sem_c):
        cid = lax.axis_index("c")
        sid = lax.axis_index("s")
        start = jnp.where(cid == 0, sid * K0, NS * K0 + sid * K1)
        ph = jnp.where(cid == 0, PH0, PH1)

        def zfill(i, carry):
            r = i // (Dh // 16)
            cc = i % (Dh // 16)
            rows_v[0, r, pl.ds(cc * 16, 16)] = jnp.zeros((16,), jnp.float32)
            return carry

        with jax.named_scope("agg_zero"):
            lax.fori_loop(0, EW * (Dh // 16), zfill, 0)
            for q in range(rps // EW):
                pltpu.sync_copy(rows_v.at[0],
                                agg_sh.at[pl.ds(sid * rps + q * EW, EW)])
        pltpu.sync_copy(z1_hbm, c_sh.at[pl.ds(sid * rps, rps)])
        pltpu.sync_copy(nd_hbm.at[pl.ds(sid * rps, rps)],
                        nd_sh.at[pl.ds(sid * rps, rps)])

        @pl.when(cid != 0)
        def _():
            pltpu.sync_copy(src_hbm.at[pl.ds(start, K1)], src_v)
            pltpu.sync_copy(dst_hbm.at[pl.ds(start, K1)], dst_v)

        plsc.subcore_barrier()
        semys = (sem_y0, sem_y1)

        def row_loop(y_hbm, base):
            def fire_y(j, b):
                pltpu.async_copy(y_hbm.at[src_v.at[base + j]], rows_v.at[b],
                                 semys[b])

            def drain_y(j, b):
                pltpu.make_async_copy(y_hbm.at[src_v.at[base + j]],
                                      rows_v.at[b], semys[b]).wait()

            for b in range(NB):
                fire_y(b, b)

            def body(tt, carry):
                for b in range(NB):
                    j = tt * NB + b
                    drain_y(j, b)
                    pltpu.sync_copy(rows_v.at[b],
                                    agg_sh.at[dst_v.at[base + j]], add=True)

                    @pl.when(j + NB < ph)
                    def _():
                        fire_y(j + NB, b)
                return carry

            lax.fori_loop(0, ph // NB, body, 0)

        for p in range(P):
            base = jnp.where(cid == 0, 0, p * PH1)

            @pl.when(cid == 0)
            def _():
                pstart = start + p * PH0
                pltpu.sync_copy(src_hbm.at[pl.ds(pstart, PH0)], src_v)
                pltpu.sync_copy(dst_hbm.at[pl.ds(pstart, PH0)], dst_v)

            def fire_nd(j, carry):
                pltpu.async_copy(nd_sh.at[dst_v.at[base + j]],
                                 ndv_v.at[base + j], sem_n)
                return carry

            lax.fori_loop(0, ph, fire_nd, 0)

            with jax.named_scope("row_loop"):
                @pl.when(cid == 0)
                def _():
                    row_loop(y0_hbm, base)

                @pl.when(cid != 0)
                def _():
                    row_loop(y1_hbm, base)

            with jax.named_scope("c_part"):
                def drain_nd(j, carry):
                    pltpu.make_async_copy(nd_sh.at[dst_v.at[base + j]],
                                          ndv_v.at[base + j], sem_n).wait()
                    return carry

                lax.fori_loop(0, ph, drain_nd, 0)

                def fire_c(j, carry):
                    pltpu.async_copy(ndv_v.at[base + j],
                                     c_sh.at[src_v.at[base + j]],
                                     sem_c, add=True)
                    return carry

                lax.fori_loop(0, ph, fire_c, 0)

                def drain_c(j, carry):
                    pltpu.make_async_copy(ndv_v.at[base + j],
                                          c_sh.at[src_v.at[base + j]],
                                          sem_c).wait()
                    return carry

                lax.fori_loop(0, ph, drain_c, 0)

        plsc.subcore_barrier()

        with jax.named_scope("agg_writeback"):
            pltpu.sync_copy(agg_sh.at[pl.ds(sid * rps, rps)],
                            agg_out.at[cid, pl.ds(sid * rps, rps)])
        pltpu.sync_copy(c_sh.at[pl.ds(sid * rps, rps)],
                        c_out.at[cid, pl.ds(sid * rps, rps)])

    return edge


def _dense1(do_p, di_p, xpad, W1, NPAD, BN):
    Din = xpad.shape[1]
    Dh = W1.shape[1]

    def body(do_ref, di_ref, x_ref, w1_ref, y_ref, y2_ref, ns_ref, nd_ref):
        deg_o = do_ref[0, :] + do_ref[1, :]
        deg_i = di_ref[0, :] + di_ref[1, :]
        ns = jnp.where(deg_o > 0, lax.rsqrt(jnp.maximum(deg_o, 1e-12)), 0.0)
        nd = jnp.where(deg_i > 0, lax.rsqrt(jnp.maximum(deg_i, 1e-12)), 0.0)
        ns_ref[0, :] = ns
        nd_ref[0, :] = nd
        yv = jnp.dot(x_ref[...], w1_ref[...],
                     preferred_element_type=jnp.float32) * ns[:, None]
        y_ref[...] = yv
        y2_ref[...] = yv

    grid = (NPAD // BN,)
    return pl.pallas_call(
        body,
        grid=grid,
        in_specs=[
            pl.BlockSpec((NC, BN), lambda i: (0, i)),
            pl.BlockSpec((NC, BN), lambda i: (0, i)),
            pl.BlockSpec((BN, Din), lambda i: (i, 0)),
            pl.BlockSpec((Din, Dh), lambda i: (0, 0)),
        ],
        out_specs=[
            pl.BlockSpec((BN, Dh), lambda i: (i, 0)),
            pl.BlockSpec((BN, Dh), lambda i: (i, 0)),
            pl.BlockSpec((1, BN), lambda i: (0, i)),
            pl.BlockSpec((1, BN), lambda i: (0, i)),
        ],
        out_shape=[
            jax.ShapeDtypeStruct((NPAD, Dh), jnp.float32),
            jax.ShapeDtypeStruct((NPAD, Dh), jnp.float32),
            jax.ShapeDtypeStruct((1, NPAD), jnp.float32),
            jax.ShapeDtypeStruct((1, NPAD), jnp.float32),
        ],
    )(do_p, di_p, xpad, W1)


def _dense2(agg_p, c_p, ns, nd, b1, W2, b2, NPAD, N, BN):
    Dh = agg_p.shape[2]
    ncls = W2.shape[1]
    grid_n = NPAD // BN

    def body(agg_ref, c_ref, ns_ref, nd_ref, b1_ref, w2_ref, b2_ref,
             out_ref, s_ref):
        i = pl.program_id(0)
        agg = agg_ref[0] + agg_ref[1]
        h1 = jnp.maximum(nd_ref[0, :][:, None] * agg + b1_ref[0, :][None, :],
                         0.0)
        w = ns_ref[0, :] * (c_ref[0, :] + c_ref[1, :])
        row = i * BN + lax.broadcasted_iota(jnp.int32, (1, BN), 1)[0]
        w = jnp.where(row < N, w, 0.0)
        part = jnp.dot(w[None, :], h1, preferred_element_type=jnp.float32)

        @pl.when(i == 0)
        def _():
            s_ref[...] = part

        @pl.when(i > 0)
        def _():
            s_ref[...] = s_ref[...] + part

        @pl.when(i == grid_n - 1)
        def _():
            out_ref[...] = jnp.dot(s_ref[...] * (1.0 / N), w2_ref[...],
                                   preferred_element_type=jnp.float32) \
                + b2_ref[...]

    return pl.pallas_call(
        body,
        grid=(grid_n,),
        in_specs=[
            pl.BlockSpec((NC, BN, Dh), lambda i: (0, i, 0)),
            pl.BlockSpec((NC, BN), lambda i: (0, i)),
            pl.BlockSpec((1, BN), lambda i: (0, i)),
            pl.BlockSpec((1, BN), lambda i: (0, i)),
            pl.BlockSpec((1, Dh), lambda i: (0, 0)),
            pl.BlockSpec((Dh, ncls), lambda i: (0, 0)),
            pl.BlockSpec((1, ncls), lambda i: (0, 0)),
        ],
        out_specs=pl.BlockSpec((1, ncls), lambda i: (0, 0)),
        out_shape=jax.ShapeDtypeStruct((1, ncls), jnp.float32),
        scratch_shapes=[pltpu.VMEM((1, Dh), jnp.float32)],
    )(agg_p, c_p, ns, nd, b1, W2, b2)


def kernel(x, edge_index, W1, b1, W2, b2):
    N, Din = x.shape
    Dh = W1.shape[1]
    E = edge_index.shape[1]
    NPAD = -(-N // 2048) * 2048
    BT = BTOT
    EPAD = BT * EW
    assert EPAD >= E

    src = edge_index[0]
    dst = edge_index[1]
    padv = jnp.full((EPAD - E,), N, jnp.int32)
    srcp = jnp.concatenate([src, padv]).reshape(BT, EW)
    dstp = jnp.concatenate([dst, padv]).reshape(BT, EW)
    xpad = jnp.pad(x, ((0, NPAD - N), (0, 0)))
    rps = NPAD // NS
    z1 = jnp.zeros((rps,), jnp.float32)
    ones = jnp.ones((EW,), jnp.float32)

    do_p, di_p = _make_hist_kernel(NPAD)(srcp, dstp, ones, z1)
    y, y2, ns, nd = _dense1(do_p, di_p, xpad, W1, NPAD, 1024)
    agg_p, c_p = _make_edge_kernel(NPAD, Dh)(
        srcp, dstp, y, y2, nd.reshape(NPAD), z1)
    out = _dense2(agg_p, c_p, ns, nd, b1.reshape(1, Dh), W2,
                  b2.reshape(1, W2.shape[1]), NPAD, N, 1024)
    return out


# speedup vs baseline: 1.0999x; 1.0999x over previous
import functools

import jax
import jax.numpy as jnp
from jax import lax
from jax.experimental import pallas as pl
from jax.experimental.pallas import tpu as pltpu
from jax.experimental.pallas import tpu_sc as plsc

NC = 2
NS = 16
EW = 128
K0 = 144
K1 = 16
BTOT = NS * (K0 + K1)
P = 6
PH0 = K0 // P
NB = 2


def _sc_mesh():
    return plsc.VectorSubcoreMesh(core_axis_name="c", subcore_axis_name="s")


def _make_hist_kernel(NPAD):
    rps = NPAD // NS
    KH = BTOT // (NC * NS)

    @functools.partial(
        pl.kernel,
        mesh=_sc_mesh(),
        out_type=(
            jax.ShapeDtypeStruct((NC, NPAD), jnp.float32),
            jax.ShapeDtypeStruct((NC, NPAD), jnp.float32),
        ),
        scratch_types=[
            pltpu.VMEM((KH, EW), jnp.int32),
            pltpu.VMEM((KH, EW), jnp.int32),
            pltpu.VMEM((EW,), jnp.float32),
            pltpu.VMEM_SHARED((NPAD,), jnp.float32),
            pltpu.VMEM_SHARED((NPAD,), jnp.float32),
        ],
    )
    def hist(src_hbm, dst_hbm, ones_hbm, z1_hbm, do_out, di_out,
             src_v, dst_v, ones_v, do_sh, di_sh):
        cid = lax.axis_index("c")
        sid = lax.axis_index("s")
        wid = sid * NC + cid
        pltpu.sync_copy(z1_hbm, do_sh.at[pl.ds(sid * rps, rps)])
        pltpu.sync_copy(z1_hbm, di_sh.at[pl.ds(sid * rps, rps)])
        pltpu.sync_copy(ones_hbm, ones_v)
        pltpu.sync_copy(src_hbm.at[pl.ds(wid * KH, KH)], src_v)
        pltpu.sync_copy(dst_hbm.at[pl.ds(wid * KH, KH)], dst_v)
        plsc.subcore_barrier()

        def body(j, carry):
            pltpu.sync_copy(ones_v, do_sh.at[src_v.at[j]], add=True)
            pltpu.sync_copy(ones_v, di_sh.at[dst_v.at[j]], add=True)
            return carry

        lax.fori_loop(0, KH, body, 0)
        plsc.subcore_barrier()
        pltpu.sync_copy(do_sh.at[pl.ds(sid * rps, rps)],
                        do_out.at[cid, pl.ds(sid * rps, rps)])
        pltpu.sync_copy(di_sh.at[pl.ds(sid * rps, rps)],
                        di_out.at[cid, pl.ds(sid * rps, rps)])

    return hist


def _make_edge_kernel(NPAD, Dh):
    rps = NPAD // NS

    @functools.partial(
        pl.kernel,
        mesh=_sc_mesh(),
        out_type=(
            jax.ShapeDtypeStruct((NC, NPAD, Dh), jnp.float32),
            jax.ShapeDtypeStruct((NC, NPAD), jnp.float32),
        ),
        scratch_types=[
            pltpu.VMEM((PH0, EW), jnp.int32),
            pltpu.VMEM((PH0, EW), jnp.int32),
            pltpu.VMEM((NB, EW, Dh), jnp.float32),
            pltpu.VMEM((PH0, EW), jnp.float32),
            pltpu.VMEM_SHARED((NPAD, Dh), jnp.float32),
            pltpu.VMEM_SHARED((NPAD,), jnp.float32),
            pltpu.VMEM_SHARED((NPAD,), jnp.float32),
            pltpu.SemaphoreType.DMA,
            pltpu.SemaphoreType.DMA,
            pltpu.SemaphoreType.DMA,
            pltpu.SemaphoreType.DMA,
        ],
    )
    def edge(src_hbm, dst_hbm, y0_hbm, y1_hbm, nd_hbm, z1_hbm,
             agg_out, c_out, src_v, dst_v, rows_v, ndv_v,
             agg_sh, c_sh, nd_sh, sem_y0, sem_y1, sem_n, sem_c):
        cid = lax.axis_index("c")
        sid = lax.axis_index("s")

        def zfill(i, carry):
            r = i // (Dh // 16)
            cc = i % (Dh // 16)
            rows_v[0, r, pl.ds(cc * 16, 16)] = jnp.zeros((16,), jnp.float32)
            return carry

        with jax.named_scope("agg_zero"):
            lax.fori_loop(0, EW * (Dh // 16), zfill, 0)
            for q in range(rps // EW):
                pltpu.sync_copy(rows_v.at[0],
                                agg_sh.at[pl.ds(sid * rps + q * EW, EW)])
        pltpu.sync_copy(z1_hbm, c_sh.at[pl.ds(sid * rps, rps)])
        pltpu.sync_copy(nd_hbm.at[pl.ds(sid * rps, rps)],
                        nd_sh.at[pl.ds(sid * rps, rps)])

        @pl.when(cid != 0)
        def _():
            pltpu.sync_copy(src_hbm.at[pl.ds(NS * K0 + sid * K1, K1)],
                            src_v.at[pl.ds(0, K1)])
            pltpu.sync_copy(dst_hbm.at[pl.ds(NS * K0 + sid * K1, K1)],
                            dst_v.at[pl.ds(0, K1)])

        plsc.subcore_barrier()
        semys = (sem_y0, sem_y1)

        def row_loop(y_hbm, n):
            def fire_y(j, b):
                pltpu.async_copy(y_hbm.at[src_v.at[j]], rows_v.at[b],
                                 semys[b])

            def drain_y(j, b):
                pltpu.make_async_copy(y_hbm.at[src_v.at[j]],
                                      rows_v.at[b], semys[b]).wait()

            for b in range(NB):
                fire_y(b, b)

            def body(tt, carry):
                for b in range(NB):
                    j = tt * NB + b
                    drain_y(j, b)
                    pltpu.sync_copy(rows_v.at[b],
                                    agg_sh.at[dst_v.at[j]], add=True)

                    @pl.when(j + NB < n)
                    def _():
                        fire_y(j + NB, b)
                return carry

            lax.fori_loop(0, n // NB, body, 0)

        def phase_work(y_hbm, n):
            def fire_nd(j, carry):
                pltpu.async_copy(nd_sh.at[dst_v.at[j]], ndv_v.at[j], sem_n)
                return carry

            lax.fori_loop(0, n, fire_nd, 0)
            with jax.named_scope("row_loop"):
                row_loop(y_hbm, n)

            with jax.named_scope("c_part"):
                def drain_nd(j, carry):
                    pltpu.make_async_copy(nd_sh.at[dst_v.at[j]],
                                          ndv_v.at[j], sem_n).wait()
                    return carry

                lax.fori_loop(0, n, drain_nd, 0)

                def fire_c(j, carry):
                    pltpu.async_copy(ndv_v.at[j], c_sh.at[src_v.at[j]],
                                     sem_c, add=True)
                    return carry

                lax.fori_loop(0, n, fire_c, 0)

                def drain_c(j, carry):
                    pltpu.make_async_copy(ndv_v.at[j], c_sh.at[src_v.at[j]],
                                          sem_c).wait()
                    return carry

                lax.fori_loop(0, n, drain_c, 0)

        for p in range(P):
            @pl.when(cid == 0)
            def _():
                pstart = sid * K0 + p * PH0
                pltpu.sync_copy(src_hbm.at[pl.ds(pstart, PH0)], src_v)
                pltpu.sync_copy(dst_hbm.at[pl.ds(pstart, PH0)], dst_v)
                phase_work(y0_hbm, PH0)

            if p == 0:
                @pl.when(cid != 0)
                def _():
                    phase_work(y1_hbm, K1)

        plsc.subcore_barrier()

        with jax.named_scope("agg_writeback"):
            pltpu.sync_copy(agg_sh.at[pl.ds(sid * rps, rps)],
                            agg_out.at[cid, pl.ds(sid * rps, rps)])
        pltpu.sync_copy(c_sh.at[pl.ds(sid * rps, rps)],
                        c_out.at[cid, pl.ds(sid * rps, rps)])

    return edge


def _dense1(do_p, di_p, xpad, W1, NPAD, BN):
    Din = xpad.shape[1]
    Dh = W1.shape[1]

    def body(do_ref, di_ref, x_ref, w1_ref, y_ref, y2_ref, ns_ref, nd_ref):
        deg_o = do_ref[0, :] + do_ref[1, :]
        deg_i = di_ref[0, :] + di_ref[1, :]
        ns = jnp.where(deg_o > 0, lax.rsqrt(jnp.maximum(deg_o, 1e-12)), 0.0)
        nd = jnp.where(deg_i > 0, lax.rsqrt(jnp.maximum(deg_i, 1e-12)), 0.0)
        ns_ref[0, :] = ns
        nd_ref[0, :] = nd
        yv = jnp.dot(x_ref[...], w1_ref[...],
                     preferred_element_type=jnp.float32) * ns[:, None]
        y_ref[...] = yv
        y2_ref[...] = yv

    grid = (NPAD // BN,)
    return pl.pallas_call(
        body,
        grid=grid,
        in_specs=[
            pl.BlockSpec((NC, BN), lambda i: (0, i)),
            pl.BlockSpec((NC, BN), lambda i: (0, i)),
            pl.BlockSpec((BN, Din), lambda i: (i, 0)),
            pl.BlockSpec((Din, Dh), lambda i: (0, 0)),
        ],
        out_specs=[
            pl.BlockSpec((BN, Dh), lambda i: (i, 0)),
            pl.BlockSpec((BN, Dh), lambda i: (i, 0)),
            pl.BlockSpec((1, BN), lambda i: (0, i)),
            pl.BlockSpec((1, BN), lambda i: (0, i)),
        ],
        out_shape=[
            jax.ShapeDtypeStruct((NPAD, Dh), jnp.float32),
            jax.ShapeDtypeStruct((NPAD, Dh), jnp.float32),
            jax.ShapeDtypeStruct((1, NPAD), jnp.float32),
            jax.ShapeDtypeStruct((1, NPAD), jnp.float32),
        ],
    )(do_p, di_p, xpad, W1)


def _dense2(agg_p, c_p, ns, nd, b1, W2, b2, NPAD, N, BN):
    Dh = agg_p.shape[2]
    ncls = W2.shape[1]
    grid_n = NPAD // BN

    def body(agg_ref, c_ref, ns_ref, nd_ref, b1_ref, w2_ref, b2_ref,
             out_ref, s_ref):
        i = pl.program_id(0)
        agg = agg_ref[0] + agg_ref[1]
        h1 = jnp.maximum(nd_ref[0, :][:, None] * agg + b1_ref[0, :][None, :],
                         0.0)
        w = ns_ref[0, :] * (c_ref[0, :] + c_ref[1, :])
        row = i * BN + lax.broadcasted_iota(jnp.int32, (1, BN), 1)[0]
        w = jnp.where(row < N, w, 0.0)
        part = jnp.dot(w[None, :], h1, preferred_element_type=jnp.float32)

        @pl.when(i == 0)
        def _():
            s_ref[...] = part

        @pl.when(i > 0)
        def _():
            s_ref[...] = s_ref[...] + part

        @pl.when(i == grid_n - 1)
        def _():
            out_ref[...] = jnp.dot(s_ref[...] * (1.0 / N), w2_ref[...],
                                   preferred_element_type=jnp.float32) \
                + b2_ref[...]

    return pl.pallas_call(
        body,
        grid=(grid_n,),
        in_specs=[
            pl.BlockSpec((NC, BN, Dh), lambda i: (0, i, 0)),
            pl.BlockSpec((NC, BN), lambda i: (0, i)),
            pl.BlockSpec((1, BN), lambda i: (0, i)),
            pl.BlockSpec((1, BN), lambda i: (0, i)),
            pl.BlockSpec((1, Dh), lambda i: (0, 0)),
            pl.BlockSpec((Dh, ncls), lambda i: (0, 0)),
            pl.BlockSpec((1, ncls), lambda i: (0, 0)),
        ],
        out_specs=pl.BlockSpec((1, ncls), lambda i: (0, 0)),
        out_shape=jax.ShapeDtypeStruct((1, ncls), jnp.float32),
        scratch_shapes=[pltpu.VMEM((1, Dh), jnp.float32)],
    )(agg_p, c_p, ns, nd, b1, W2, b2)


def kernel(x, edge_index, W1, b1, W2, b2):
    N, Din = x.shape
    Dh = W1.shape[1]
    E = edge_index.shape[1]
    NPAD = -(-N // 2048) * 2048
    BT = BTOT
    EPAD = BT * EW
    assert EPAD >= E

    src = edge_index[0]
    dst = edge_index[1]
    padv = jnp.full((EPAD - E,), N, jnp.int32)
    srcp = jnp.concatenate([src, padv]).reshape(BT, EW)
    dstp = jnp.concatenate([dst, padv]).reshape(BT, EW)
    xpad = jnp.pad(x, ((0, NPAD - N), (0, 0)))
    rps = NPAD // NS
    z1 = jnp.zeros((rps,), jnp.float32)
    ones = jnp.ones((EW,), jnp.float32)

    do_p, di_p = _make_hist_kernel(NPAD)(srcp, dstp, ones, z1)
    y, y2, ns, nd = _dense1(do_p, di_p, xpad, W1, NPAD, 1024)
    agg_p, c_p = _make_edge_kernel(NPAD, Dh)(
        srcp, dstp, y, y2, nd.reshape(NPAD), z1)
    out = _dense2(agg_p, c_p, ns, nd, b1.reshape(1, Dh), W2,
                  b2.reshape(1, W2.shape[1]), NPAD, N, 1024)
    return out
